# baseline (speedup 1.0000x reference)
import functools

import jax
import jax.numpy as jnp
from jax import lax
from jax.experimental import pallas as pl
from jax.experimental.pallas import tpu as pltpu
from jax.experimental.pallas import tpu_sc as plsc

BATCH = 1024
STACK = 100000
FEAT = 128
KEEP = STACK - BATCH

NC = 2
NS = 16
NW = NC * NS

KEEP_E = KEEP * FEAT
SHIFT_E = BATCH * FEAT
SPAN_E = KEEP_E // NW
CHUNK_E = 63480
NBUF = 2
NFULL = SPAN_E // CHUNK_E
TAIL_E = SPAN_E - NFULL * CHUNK_E
XB_E = BATCH * FEAT // NW

_mesh = plsc.VectorSubcoreMesh(core_axis_name="c", subcore_axis_name="s")


@functools.partial(
    pl.kernel,
    out_type=jax.ShapeDtypeStruct((STACK * FEAT,), jnp.float32),
    mesh=_mesh,
    scratch_types=(
        [pltpu.VMEM((CHUNK_E,), jnp.float32) for _ in range(NBUF)]
        + [pltpu.VMEM((XB_E,), jnp.float32)]
        + [pltpu.SemaphoreType.DMA for _ in range(2 * NBUF + 1)]
    ),
    compiler_params=pltpu.CompilerParams(
        disable_bounds_checks=True,
        disable_semaphore_checks=True,
        skip_device_barrier=True,
    ),
)
def _sc_shift(x_hbm, st_hbm, out_hbm, *scratch):
    bufs = scratch[:NBUF]
    bx = scratch[NBUF]
    rsem = scratch[NBUF + 1 : 2 * NBUF + 1]
    wsem = scratch[2 * NBUF + 1 : 3 * NBUF + 1]
    sx = scratch[3 * NBUF + 1]
    b0 = bufs[0]
    rs0 = rsem[0]
    wid = lax.axis_index("s") * NC + lax.axis_index("c")
    src_base = SHIFT_E + wid * SPAN_E
    dst_base = wid * SPAN_E

    def rd(i):
        return pltpu.make_async_copy(
            st_hbm.at[pl.ds(src_base + i * CHUNK_E, CHUNK_E)],
            bufs[i % NBUF],
            rsem[i % NBUF],
        )

    def wr(i):
        return pltpu.make_async_copy(
            bufs[i % NBUF],
            out_hbm.at[pl.ds(dst_base + i * CHUNK_E, CHUNK_E)],
            wsem[i % NBUF],
        )

    x_rd = pltpu.make_async_copy(x_hbm.at[pl.ds(wid * XB_E, XB_E)], bx, sx)
    x_rd.start()

    for i in range(NBUF - 1):
        rd(i).start()
    for i in range(NFULL):
        if i + NBUF - 1 < NFULL:
            if i >= 1:
                wr(i - 1).wait()
            rd(i + NBUF - 1).start()
        rd(i).wait()
        wr(i).start()

    _last_b0 = ((NFULL - 1) // NBUF) * NBUF
    for i in range(NFULL - NBUF, _last_b0 + 1):
        wr(i).wait()
    tail_rd = pltpu.make_async_copy(
        st_hbm.at[pl.ds(src_base + NFULL * CHUNK_E, TAIL_E)],
        b0.at[pl.ds(0, TAIL_E)],
        rs0,
    )
    tail_rd.start()
    x_rd.wait()
    x_wr = pltpu.make_async_copy(bx, out_hbm.at[pl.ds(KEEP_E + wid * XB_E, XB_E)], sx)
    x_wr.start()
    tail_rd.wait()
    tail_wr = pltpu.make_async_copy(
        b0.at[pl.ds(0, TAIL_E)],
        out_hbm.at[pl.ds(dst_base + NFULL * CHUNK_E, TAIL_E)],
        rs0,
    )
    tail_wr.start()
    for i in range(_last_b0 + 1, NFULL):
        wr(i).wait()
    x_wr.wait()
    tail_wr.wait()


def kernel(x, latent_stack):
    flat = _sc_shift(x.reshape(-1), latent_stack.reshape(-1))
    return flat.reshape(STACK, FEAT)


# speedup vs baseline: 1.0030x; 1.0030x over previous
import functools

import jax
import jax.numpy as jnp
from jax import lax
from jax.experimental import pallas as pl
from jax.experimental.pallas import tpu as pltpu
from jax.experimental.pallas import tpu_sc as plsc

BATCH = 1024
STACK = 100000
FEAT = 128
KEEP = STACK - BATCH

NC = 2
NS = 16
NW = NC * NS

KEEP_E = KEEP * FEAT
SHIFT_E = BATCH * FEAT
SPAN_E = KEEP_E // NW
NBUF = 2
XB_E = BATCH * FEAT // NW

CHUNK_E = 60624
WARM_E = SPAN_E - 6 * CHUNK_E
SIZES = (WARM_E,) + (CHUNK_E,) * 6
OFFS = tuple(sum(SIZES[:i]) for i in range(len(SIZES)))
NFULL = len(SIZES)

_mesh = plsc.VectorSubcoreMesh(core_axis_name="c", subcore_axis_name="s")


@functools.partial(
    pl.kernel,
    out_type=jax.ShapeDtypeStruct((STACK * FEAT,), jnp.float32),
    mesh=_mesh,
    scratch_types=(
        [pltpu.VMEM((CHUNK_E,), jnp.float32) for _ in range(NBUF)]
        + [pltpu.VMEM((XB_E,), jnp.float32)]
        + [pltpu.SemaphoreType.DMA for _ in range(2 * NBUF + 1)]
    ),
    compiler_params=pltpu.CompilerParams(
        disable_bounds_checks=True,
        disable_semaphore_checks=True,
        skip_device_barrier=True,
    ),
)
def _sc_shift(x_hbm, st_hbm, out_hbm, *scratch):
    bufs = scratch[:NBUF]
    bx = scratch[NBUF]
    rsem = scratch[NBUF + 1 : 2 * NBUF + 1]
    wsem = scratch[2 * NBUF + 1 : 3 * NBUF + 1]
    sx = scratch[3 * NBUF + 1]
    wid = lax.axis_index("s") * NC + lax.axis_index("c")
    src_base = SHIFT_E + wid * SPAN_E
    dst_base = wid * SPAN_E

    def rd(i):
        return pltpu.make_async_copy(
            st_hbm.at[pl.ds(src_base + OFFS[i], SIZES[i])],
            bufs[i % NBUF].at[pl.ds(0, SIZES[i])],
            rsem[i % NBUF],
        )

    def wr(i):
        return pltpu.make_async_copy(
            bufs[i % NBUF].at[pl.ds(0, SIZES[i])],
            out_hbm.at[pl.ds(dst_base + OFFS[i], SIZES[i])],
            wsem[i % NBUF],
        )

    x_rd = pltpu.make_async_copy(x_hbm.at[pl.ds(wid * XB_E, XB_E)], bx, sx)
    x_rd.start()

    for i in range(NBUF - 1):
        rd(i).start()
    for i in range(NFULL):
        if i + NBUF - 1 < NFULL:
            if i >= 1:
                wr(i - 1).wait()
            rd(i + NBUF - 1).start()
        rd(i).wait()
        wr(i).start()

    x_rd.wait()
    x_wr = pltpu.make_async_copy(bx, out_hbm.at[pl.ds(KEEP_E + wid * XB_E, XB_E)], sx)
    x_wr.start()
    for i in range(NFULL - NBUF, NFULL):
        wr(i).wait()
    x_wr.wait()


def kernel(x, latent_stack):
    flat = _sc_shift(x.reshape(-1), latent_stack.reshape(-1))
    return flat.reshape(STACK, FEAT)
